# Initial kernel scaffold; baseline (speedup 1.0000x reference)
import jax
import jax.numpy as jnp
from jax.experimental import pallas as pl


def kernel(x, edge_index, W0, b0, g0, be0, Wl1, bl1, Wr1, g1, be1, Wl2, bl2, Wr2, g2, be2, W1, b1, g3, be3, W2, b2):
    raise NotImplementedError("write your pallas kernel here")


# speedup vs baseline: 2.4482x; 2.4482x over previous
import functools

import jax
import jax.numpy as jnp
from jax import lax
from jax.experimental import pallas as pl
from jax.experimental.pallas import tpu as pltpu
from jax.experimental.pallas import tpu_sc as plsc

N, E, D, H = 10000, 160000, 1024, 128

NC, NS = 2, 16
NW = NC * NS
CHUNK = 128
EPAD = 163840
CPW = EPAD // NW
NCHUNK = CPW // CHUNK
NPAD = 10240
CW = 128
RB = 400
GRID = N // RB

_MESH = plsc.VectorSubcoreMesh(core_axis_name="c", subcore_axis_name="s",
                               num_cores=NC, num_subcores=NS)


def _sc_segsum(h, src, dst, zrow):
    @functools.partial(
        pl.kernel,
        out_type=jax.ShapeDtypeStruct((NC, NPAD, H), jnp.float32),
        mesh=_MESH,
        scratch_types=[
            pltpu.VMEM_SHARED((NPAD, H), jnp.float32),
            pltpu.VMEM((CHUNK,), jnp.int32),
            pltpu.VMEM((CHUNK,), jnp.int32),
            pltpu.VMEM((CHUNK, H), jnp.float32),
            pltpu.SemaphoreType.DMA,
        ],
    )
    def k(h_hbm, src_hbm, dst_hbm, zrow_hbm,
          agg_out, agg_sh, src_v, dst_v, rows_v, sem):
        cid = lax.axis_index("c")
        sid = lax.axis_index("s")
        wid = sid * NC + cid
        rpt = NPAD // NS
        r0 = sid * rpt
        for j in range(rpt // CHUNK):
            sl = pl.ds(r0 + j * CHUNK, CHUNK)
            pltpu.sync_copy(zrow_hbm.at[sl], rows_v)
            pltpu.sync_copy(rows_v, agg_sh.at[sl])
        plsc.subcore_barrier()

        base = wid * CPW

        def body(i, carry):
            off = pl.multiple_of(base + i * CHUNK, CHUNK)
            pltpu.sync_copy(src_hbm.at[pl.ds(off, CHUNK)], src_v)
            pltpu.sync_copy(dst_hbm.at[pl.ds(off, CHUNK)], dst_v)
            pltpu.async_copy(h_hbm.at[src_v], rows_v, sem).wait()
            pltpu.sync_copy(rows_v, agg_sh.at[dst_v], add=True)
            return carry

        lax.fori_loop(0, NCHUNK, body, 0)
        plsc.subcore_barrier()
        for j in range(rpt // CHUNK):
            sl = pl.ds(r0 + j * CHUNK, CHUNK)
            pltpu.sync_copy(agg_sh.at[sl], rows_v)
            pltpu.sync_copy(rows_v, agg_out.at[cid, sl])

    return k(h, src, dst, zrow)


def _sc_count(dst, zrow, ones_c):
    @functools.partial(
        pl.kernel,
        out_type=jax.ShapeDtypeStruct((NC, NPAD, CW), jnp.float32),
        mesh=_MESH,
        scratch_types=[
            pltpu.VMEM_SHARED((NPAD, CW), jnp.float32),
            pltpu.VMEM((CHUNK,), jnp.int32),
            pltpu.VMEM((CHUNK, CW), jnp.float32),
        ],
    )
    def k(dst_hbm, zrow_hbm, one_hbm,
          cnt_out, cnt_sh, dst_v, ones_v):
        cid = lax.axis_index("c")
        sid = lax.axis_index("s")
        wid = sid * NC + cid
        rpt = NPAD // NS
        r0 = sid * rpt
        for j in range(rpt // CHUNK):
            sl = pl.ds(r0 + j * CHUNK, CHUNK)
            pltpu.sync_copy(zrow_hbm.at[sl], ones_v)
            pltpu.sync_copy(ones_v, cnt_sh.at[sl])
        pltpu.sync_copy(one_hbm, ones_v)
        plsc.subcore_barrier()

        base = wid * CPW

        def body(i, carry):
            off = pl.multiple_of(base + i * CHUNK, CHUNK)
            pltpu.sync_copy(dst_hbm.at[pl.ds(off, CHUNK)], dst_v)
            pltpu.sync_copy(ones_v, cnt_sh.at[dst_v], add=True)
            return carry

        lax.fori_loop(0, NCHUNK, body, 0)
        plsc.subcore_barrier()
        for j in range(rpt // CHUNK):
            sl = pl.ds(r0 + j * CHUNK, CHUNK)
            pltpu.sync_copy(cnt_sh.at[sl], ones_v)
            pltpu.sync_copy(ones_v, cnt_out.at[cid, sl])

    return k(dst, zrow, ones_c)


def _ln_relu(x, g, b):
    m = jnp.mean(x, axis=-1, keepdims=True)
    v = jnp.mean((x - m) ** 2, axis=-1, keepdims=True)
    return jnp.maximum((x - m) * lax.rsqrt(v + 1e-5) * g + b, 0.0)


def _proj_body(x_ref, w_ref, b_ref, g_ref, be_ref, o_ref):
    h = jnp.dot(x_ref[...], w_ref[...], preferred_element_type=jnp.float32)
    o_ref[...] = _ln_relu(h + b_ref[...], g_ref[...], be_ref[...])


def _proj(x, w, b, g, be):
    return pl.pallas_call(
        _proj_body,
        grid=(GRID,),
        in_specs=[
            pl.BlockSpec((RB, D), lambda i: (i, 0)),
            pl.BlockSpec((D, H), lambda i: (0, 0)),
            pl.BlockSpec((1, H), lambda i: (0, 0)),
            pl.BlockSpec((1, H), lambda i: (0, 0)),
            pl.BlockSpec((1, H), lambda i: (0, 0)),
        ],
        out_specs=pl.BlockSpec((RB, H), lambda i: (i, 0)),
        out_shape=jax.ShapeDtypeStruct((N, H), jnp.float32),
    )(x, w, b, g, be)


def _sage_block(p_ref, c_ref, h_ref, wl_ref, bl_ref, wr_ref, g_ref, be_ref):
    agg = p_ref[0] + p_ref[1]
    cnt = c_ref[0][:, :1] + c_ref[1][:, :1]
    mean = agg / jnp.maximum(cnt, 1.0)
    h = h_ref[...]
    hn = (jnp.dot(mean, wl_ref[...], preferred_element_type=jnp.float32)
          + bl_ref[...]
          + jnp.dot(h, wr_ref[...], preferred_element_type=jnp.float32))
    return h + _ln_relu(hn, g_ref[...], be_ref[...])


def _combine_body(p_ref, c_ref, h_ref, wl_ref, bl_ref, wr_ref, g_ref, be_ref,
                  o_ref):
    o_ref[...] = _sage_block(p_ref, c_ref, h_ref, wl_ref, bl_ref, wr_ref,
                             g_ref, be_ref)


_SAGE_SPECS = [
    pl.BlockSpec((NC, RB, H), lambda i: (0, i, 0)),
    pl.BlockSpec((NC, RB, CW), lambda i: (0, i, 0)),
    pl.BlockSpec((RB, H), lambda i: (i, 0)),
    pl.BlockSpec((H, H), lambda i: (0, 0)),
    pl.BlockSpec((1, H), lambda i: (0, 0)),
    pl.BlockSpec((H, H), lambda i: (0, 0)),
    pl.BlockSpec((1, H), lambda i: (0, 0)),
    pl.BlockSpec((1, H), lambda i: (0, 0)),
]


def _combine(p, c, h, wl, bl, wr, g, be):
    return pl.pallas_call(
        _combine_body,
        grid=(GRID,),
        in_specs=_SAGE_SPECS,
        out_specs=pl.BlockSpec((RB, H), lambda i: (i, 0)),
        out_shape=jax.ShapeDtypeStruct((N, H), jnp.float32),
    )(p, c, h, wl, bl, wr, g, be)


def _final_body(p_ref, c_ref, h_ref, wl_ref, bl_ref, wr_ref, g_ref, be_ref,
                w1_ref, b1_ref, g3_ref, be3_ref, w2_ref, b2_ref,
                o_ref, sum_sc, max_sc):
    i = pl.program_id(0)
    h2 = _sage_block(p_ref, c_ref, h_ref, wl_ref, bl_ref, wr_ref, g_ref,
                     be_ref)
    blk = h2.reshape(RB // 8, 8, H)
    bsum = jnp.sum(blk, axis=0)
    bmax = jnp.max(blk, axis=0)

    @pl.when(i == 0)
    def _():
        sum_sc[...] = bsum
        max_sc[...] = bmax

    @pl.when(i > 0)
    def _():
        sum_sc[...] = sum_sc[...] + bsum
        max_sc[...] = jnp.maximum(max_sc[...], bmax)

    @pl.when(i == pl.num_programs(0) - 1)
    def _():
        hm = jnp.sum(sum_sc[...], axis=0, keepdims=True) / N
        hx = jnp.max(max_sc[...], axis=0, keepdims=True)
        r = jnp.concatenate([hm, hx], axis=-1)
        r8 = jnp.broadcast_to(r, (8, 2 * H))
        r8 = jnp.dot(r8, w1_ref[...], preferred_element_type=jnp.float32)
        r8 = _ln_relu(r8 + b1_ref[...], g3_ref[...], be3_ref[...])
        out8 = (jnp.dot(r8, w2_ref[...], preferred_element_type=jnp.float32)
                + b2_ref[...])
        o_ref[...] = out8[:1]


def _final(p, c, h, wl, bl, wr, g, be, w1, b1, g3, be3, w2, b2):
    return pl.pallas_call(
        _final_body,
        grid=(GRID,),
        in_specs=_SAGE_SPECS + [
            pl.BlockSpec((2 * H, H), lambda i: (0, 0)),
            pl.BlockSpec((1, H), lambda i: (0, 0)),
            pl.BlockSpec((1, H), lambda i: (0, 0)),
            pl.BlockSpec((1, H), lambda i: (0, 0)),
            pl.BlockSpec((H, H), lambda i: (0, 0)),
            pl.BlockSpec((1, H), lambda i: (0, 0)),
        ],
        out_specs=pl.BlockSpec((1, H), lambda i: (0, 0)),
        out_shape=jax.ShapeDtypeStruct((1, H), jnp.float32),
        scratch_shapes=[pltpu.VMEM((8, H), jnp.float32),
                        pltpu.VMEM((8, H), jnp.float32)],
    )(p, c, h, wl, bl, wr, g, be, w1, b1, g3, be3, w2, b2)


def kernel(x, edge_index, W0, b0, g0, be0, Wl1, bl1, Wr1, g1, be1,
           Wl2, bl2, Wr2, g2, be2, W1, b1, g3, be3, W2, b2):
    pad = EPAD - E
    src = jnp.concatenate([edge_index[0], jnp.zeros((pad,), jnp.int32)])
    dst = jnp.concatenate([edge_index[1], jnp.full((pad,), N, jnp.int32)])
    zrow = jnp.zeros((NPAD, H), jnp.float32)
    ones_c = jnp.ones((CHUNK, CW), jnp.float32)

    r2 = lambda a: a.reshape(1, -1)

    c = _sc_count(dst, zrow, ones_c)
    h = _proj(x, W0, r2(b0), r2(g0), r2(be0))
    p1 = _sc_segsum(h, src, dst, zrow)
    h = _combine(p1, c, h, Wl1, r2(bl1), Wr1, r2(g1), r2(be1))
    p2 = _sc_segsum(h, src, dst, zrow)
    return _final(p2, c, h, Wl2, r2(bl2), Wr2, r2(g2), r2(be2),
                  W1, r2(b1), g3.reshape(1, -1), r2(be3), W2, r2(b2))


# speedup vs baseline: 3.0958x; 1.2645x over previous
import functools

import jax
import jax.numpy as jnp
from jax import lax
from jax.experimental import pallas as pl
from jax.experimental.pallas import tpu as pltpu
from jax.experimental.pallas import tpu_sc as plsc

N, E, D, H = 10000, 160000, 1024, 128

NC, NS = 2, 16
NW = NC * NS
CHUNK = 128
EPAD = 163840
CPW = EPAD // NW
NCHUNK = CPW // CHUNK
NPAD = 10240
CW = 128
RB = 400
GRID = N // RB

_MESH = plsc.VectorSubcoreMesh(core_axis_name="c", subcore_axis_name="s",
                               num_cores=NC, num_subcores=NS)


NBUF = 2


def _sc_segsum(h, src2, dst2, zrow):
    @functools.partial(
        pl.kernel,
        out_type=jax.ShapeDtypeStruct((NC, NPAD, H), jnp.float32),
        mesh=_MESH,
        scratch_types=[
            pltpu.VMEM_SHARED((NPAD, H), jnp.float32),
            pltpu.VMEM((NCHUNK, CHUNK), jnp.int32),
            pltpu.VMEM((NCHUNK, CHUNK), jnp.int32),
        ] + [pltpu.VMEM((CHUNK, H), jnp.float32) for _ in range(NBUF)]
          + [pltpu.SemaphoreType.DMA for _ in range(NBUF)],
    )
    def k(h_hbm, src_hbm, dst_hbm, zrow_hbm, agg_out, agg_sh, src_v, dst_v,
          *bufsem):
        bufs, sems = bufsem[:NBUF], bufsem[NBUF:]
        cid = lax.axis_index("c")
        sid = lax.axis_index("s")
        wid = sid * NC + cid
        rpt = NPAD // NS
        r0 = sid * rpt
        cbase = wid * NCHUNK
        pltpu.sync_copy(src_hbm.at[pl.ds(cbase, NCHUNK)], src_v)
        pltpu.sync_copy(dst_hbm.at[pl.ds(cbase, NCHUNK)], dst_v)
        pltpu.sync_copy(zrow_hbm, bufs[0])
        for j in range(rpt // CHUNK):
            pltpu.sync_copy(bufs[0], agg_sh.at[pl.ds(r0 + j * CHUNK, CHUNK)])
        for b in range(NBUF):
            pltpu.async_copy(h_hbm.at[src_v.at[b]], bufs[b], sems[b])
        plsc.subcore_barrier()

        def round_(g, carry):
            for b in range(NBUF):
                i = g * NBUF + b
                pltpu.make_async_copy(h_hbm.at[src_v.at[i]], bufs[b],
                                      sems[b]).wait()
                pltpu.sync_copy(bufs[b], agg_sh.at[dst_v.at[i]], add=True)
                j = i + NBUF

                @pl.when(j < NCHUNK)
                def _():
                    pltpu.async_copy(h_hbm.at[src_v.at[j]], bufs[b], sems[b])
            return carry

        lax.fori_loop(0, NCHUNK // NBUF, round_, 0)
        plsc.subcore_barrier()
        nout = rpt // CHUNK
        for j in range(nout):
            b = j % NBUF
            sl = pl.ds(r0 + j * CHUNK, CHUNK)
            if j >= NBUF:
                psl = pl.ds(r0 + (j - NBUF) * CHUNK, CHUNK)
                pltpu.make_async_copy(bufs[b], agg_out.at[cid, psl],
                                      sems[b]).wait()
            pltpu.sync_copy(agg_sh.at[sl], bufs[b])
            pltpu.async_copy(bufs[b], agg_out.at[cid, sl], sems[b])
        for j in range(max(0, nout - NBUF), nout):
            b = j % NBUF
            sl = pl.ds(r0 + j * CHUNK, CHUNK)
            pltpu.make_async_copy(bufs[b], agg_out.at[cid, sl],
                                  sems[b]).wait()

    return k(h, src2, dst2, zrow)


def _sc_count(dst2, zrow, ones_c):
    @functools.partial(
        pl.kernel,
        out_type=jax.ShapeDtypeStruct((NC, NPAD, CW), jnp.float32),
        mesh=_MESH,
        scratch_types=[
            pltpu.VMEM_SHARED((NPAD, CW), jnp.float32),
            pltpu.VMEM((NCHUNK, CHUNK), jnp.int32),
            pltpu.VMEM((CHUNK, CW), jnp.float32),
            pltpu.VMEM((CHUNK, CW), jnp.float32),
            pltpu.SemaphoreType.DMA,
        ],
    )
    def k(dst_hbm, zrow_hbm, one_hbm,
          cnt_out, cnt_sh, dst_v, ones_v, zbuf, sem):
        cid = lax.axis_index("c")
        sid = lax.axis_index("s")
        wid = sid * NC + cid
        rpt = NPAD // NS
        r0 = sid * rpt
        pltpu.sync_copy(dst_hbm.at[pl.ds(wid * NCHUNK, NCHUNK)], dst_v)
        pltpu.sync_copy(one_hbm, ones_v)
        pltpu.sync_copy(zrow_hbm, zbuf)
        for j in range(rpt // CHUNK):
            pltpu.sync_copy(zbuf, cnt_sh.at[pl.ds(r0 + j * CHUNK, CHUNK)])
        plsc.subcore_barrier()

        def body(i, carry):
            pltpu.sync_copy(ones_v, cnt_sh.at[dst_v.at[i]], add=True)
            return carry

        lax.fori_loop(0, NCHUNK, body, 0)
        plsc.subcore_barrier()
        nout = rpt // CHUNK
        for j in range(nout):
            sl = pl.ds(r0 + j * CHUNK, CHUNK)
            if j > 0:
                psl = pl.ds(r0 + (j - 1) * CHUNK, CHUNK)
                pltpu.make_async_copy(zbuf, cnt_out.at[cid, psl], sem).wait()
            pltpu.sync_copy(cnt_sh.at[sl], zbuf)
            pltpu.async_copy(zbuf, cnt_out.at[cid, sl], sem)
        pltpu.make_async_copy(
            zbuf, cnt_out.at[cid, pl.ds(r0 + (nout - 1) * CHUNK, CHUNK)],
            sem).wait()

    return k(dst2, zrow, ones_c)


def _ln_relu(x, g, b):
    m = jnp.mean(x, axis=-1, keepdims=True)
    v = jnp.mean((x - m) ** 2, axis=-1, keepdims=True)
    return jnp.maximum((x - m) * lax.rsqrt(v + 1e-5) * g + b, 0.0)


def _proj_body(x_ref, w_ref, b_ref, g_ref, be_ref, o_ref):
    h = jnp.dot(x_ref[...], w_ref[...], preferred_element_type=jnp.float32)
    o_ref[...] = _ln_relu(h + b_ref[...], g_ref[...], be_ref[...])


def _proj(x, w, b, g, be):
    return pl.pallas_call(
        _proj_body,
        grid=(GRID,),
        in_specs=[
            pl.BlockSpec((RB, D), lambda i: (i, 0)),
            pl.BlockSpec((D, H), lambda i: (0, 0)),
            pl.BlockSpec((1, H), lambda i: (0, 0)),
            pl.BlockSpec((1, H), lambda i: (0, 0)),
            pl.BlockSpec((1, H), lambda i: (0, 0)),
        ],
        out_specs=pl.BlockSpec((RB, H), lambda i: (i, 0)),
        out_shape=jax.ShapeDtypeStruct((N, H), jnp.float32),
    )(x, w, b, g, be)


def _sage_block(p_ref, c_ref, h_ref, wl_ref, bl_ref, wr_ref, g_ref, be_ref):
    agg = p_ref[0] + p_ref[1]
    cnt = c_ref[0][:, :1] + c_ref[1][:, :1]
    mean = agg / jnp.maximum(cnt, 1.0)
    h = h_ref[...]
    hn = (jnp.dot(mean, wl_ref[...], preferred_element_type=jnp.float32)
          + bl_ref[...]
          + jnp.dot(h, wr_ref[...], preferred_element_type=jnp.float32))
    return h + _ln_relu(hn, g_ref[...], be_ref[...])


def _combine_body(p_ref, c_ref, h_ref, wl_ref, bl_ref, wr_ref, g_ref, be_ref,
                  o_ref):
    o_ref[...] = _sage_block(p_ref, c_ref, h_ref, wl_ref, bl_ref, wr_ref,
                             g_ref, be_ref)


_SAGE_SPECS = [
    pl.BlockSpec((NC, RB, H), lambda i: (0, i, 0)),
    pl.BlockSpec((NC, RB, CW), lambda i: (0, i, 0)),
    pl.BlockSpec((RB, H), lambda i: (i, 0)),
    pl.BlockSpec((H, H), lambda i: (0, 0)),
    pl.BlockSpec((1, H), lambda i: (0, 0)),
    pl.BlockSpec((H, H), lambda i: (0, 0)),
    pl.BlockSpec((1, H), lambda i: (0, 0)),
    pl.BlockSpec((1, H), lambda i: (0, 0)),
]


def _combine(p, c, h, wl, bl, wr, g, be):
    return pl.pallas_call(
        _combine_body,
        grid=(GRID,),
        in_specs=_SAGE_SPECS,
        out_specs=pl.BlockSpec((RB, H), lambda i: (i, 0)),
        out_shape=jax.ShapeDtypeStruct((N, H), jnp.float32),
    )(p, c, h, wl, bl, wr, g, be)


def _final_body(p_ref, c_ref, h_ref, wl_ref, bl_ref, wr_ref, g_ref, be_ref,
                w1_ref, b1_ref, g3_ref, be3_ref, w2_ref, b2_ref,
                o_ref, sum_sc, max_sc):
    i = pl.program_id(0)
    h2 = _sage_block(p_ref, c_ref, h_ref, wl_ref, bl_ref, wr_ref, g_ref,
                     be_ref)
    blk = h2.reshape(RB // 8, 8, H)
    bsum = jnp.sum(blk, axis=0)
    bmax = jnp.max(blk, axis=0)

    @pl.when(i == 0)
    def _():
        sum_sc[...] = bsum
        max_sc[...] = bmax

    @pl.when(i > 0)
    def _():
        sum_sc[...] = sum_sc[...] + bsum
        max_sc[...] = jnp.maximum(max_sc[...], bmax)

    @pl.when(i == pl.num_programs(0) - 1)
    def _():
        hm = jnp.sum(sum_sc[...], axis=0, keepdims=True) / N
        hx = jnp.max(max_sc[...], axis=0, keepdims=True)
        r = jnp.concatenate([hm, hx], axis=-1)
        r8 = jnp.broadcast_to(r, (8, 2 * H))
        r8 = jnp.dot(r8, w1_ref[...], preferred_element_type=jnp.float32)
        r8 = _ln_relu(r8 + b1_ref[...], g3_ref[...], be3_ref[...])
        out8 = (jnp.dot(r8, w2_ref[...], preferred_element_type=jnp.float32)
                + b2_ref[...])
        o_ref[...] = out8[:1]


def _final(p, c, h, wl, bl, wr, g, be, w1, b1, g3, be3, w2, b2):
    return pl.pallas_call(
        _final_body,
        grid=(GRID,),
        in_specs=_SAGE_SPECS + [
            pl.BlockSpec((2 * H, H), lambda i: (0, 0)),
            pl.BlockSpec((1, H), lambda i: (0, 0)),
            pl.BlockSpec((1, H), lambda i: (0, 0)),
            pl.BlockSpec((1, H), lambda i: (0, 0)),
            pl.BlockSpec((H, H), lambda i: (0, 0)),
            pl.BlockSpec((1, H), lambda i: (0, 0)),
        ],
        out_specs=pl.BlockSpec((1, H), lambda i: (0, 0)),
        out_shape=jax.ShapeDtypeStruct((1, H), jnp.float32),
        scratch_shapes=[pltpu.VMEM((8, H), jnp.float32),
                        pltpu.VMEM((8, H), jnp.float32)],
    )(p, c, h, wl, bl, wr, g, be, w1, b1, g3, be3, w2, b2)


def kernel(x, edge_index, W0, b0, g0, be0, Wl1, bl1, Wr1, g1, be1,
           Wl2, bl2, Wr2, g2, be2, W1, b1, g3, be3, W2, b2):
    pad = EPAD - E
    src2 = jnp.concatenate([edge_index[0], jnp.zeros((pad,), jnp.int32)]
                           ).reshape(EPAD // CHUNK, CHUNK)
    dst2 = jnp.concatenate([edge_index[1], jnp.full((pad,), N, jnp.int32)]
                           ).reshape(EPAD // CHUNK, CHUNK)
    zrow = jnp.zeros((CHUNK, H), jnp.float32)
    ones_c = jnp.ones((CHUNK, CW), jnp.float32)

    r2 = lambda a: a.reshape(1, -1)

    c = _sc_count(dst2, zrow, ones_c)
    h = _proj(x, W0, r2(b0), r2(g0), r2(be0))
    p1 = _sc_segsum(h, src2, dst2, zrow)
    h = _combine(p1, c, h, Wl1, r2(bl1), Wr1, r2(g1), r2(be1))
    p2 = _sc_segsum(h, src2, dst2, zrow)
    return _final(p2, c, h, Wl2, r2(bl2), Wr2, r2(g2), r2(be2),
                  W1, r2(b1), g3.reshape(1, -1), r2(be3), W2, r2(b2))
